# baseline (speedup 1.0000x reference)
import functools

import jax
import jax.numpy as jnp
from jax import lax
from jax.experimental import pallas as pl
from jax.experimental.pallas import tpu as pltpu
from jax.experimental.pallas import tpu_sc as plsc

B = 4096
NUM_NEG = 128
NUM_ENT = 100000
NUM_REL = 500
D = 128
DW = D // 2
GAMMA = 24.0

NC = 2
NS = 16
NW = NC * NS
BW = B // NW
PACK_CHUNK = 125
ENT_PER_WORKER = NUM_ENT // NW


def _body(q0_hbm, q1_hbm, pos_hbm, neg_hbm, ent_hbm, rel_hbm,
          pout_hbm, nout_hbm, entp_hbm, relp_hbm, flags_hbm,
          idxq0_v, idxq1_v, idxpos_v, idxneg_v,
          center_v, rel_v, negbuf0, negbuf1, out_v, poslog_v, trans_v,
          packin_v, packout_v, flagio_v, allf_v,
          semg, sem0, sem1):
    cc = lax.axis_index("c")
    sid = lax.axis_index("s")
    wid = sid * NC + cc
    base = wid * BW
    lane = lax.iota(jnp.int32, 16)

    flagio_v[pl.ds(0, 16)] = jnp.zeros((16,), jnp.int32)
    pltpu.sync_copy(flagio_v, flags_hbm.at[pl.ds(wid * 16, 16)])

    pltpu.sync_copy(neg_hbm.at[pl.ds(base, BW)], idxneg_v)
    pltpu.sync_copy(q0_hbm.at[pl.ds(base, BW)], idxq0_v)
    pltpu.sync_copy(q1_hbm.at[pl.ds(base, BW)], idxq1_v)
    pltpu.sync_copy(pos_hbm.at[pl.ds(base, BW)], idxpos_v)

    def pack_rows(src_hbm, dst_hbm, src_row0, dst_row0):
        pltpu.sync_copy(src_hbm.at[pl.ds(src_row0, PACK_CHUNK)], packin_v)

        def row_body(r, carry):
            for c in range(4):
                a = packin_v[r, pl.ds(c * 32, 16)]
                b = packin_v[r, pl.ds(c * 32 + 16, 16)]
                p = plsc.pack(a, b, format=plsc.PackFormat.INTERLEAVED)
                packout_v[r, pl.ds(c * 16, 16)] = plsc.bitcast(p, jnp.int32)
            return carry
        lax.fori_loop(0, PACK_CHUNK, row_body, 0)
        pltpu.sync_copy(packout_v, dst_hbm.at[pl.ds(dst_row0, PACK_CHUNK)])

    ent_row0 = wid * ENT_PER_WORKER

    def pack_body(i, carry):
        r0 = ent_row0 + i * PACK_CHUNK
        pack_rows(ent_hbm, entp_hbm, r0, r0)
        return carry
    lax.fori_loop(0, ENT_PER_WORKER // PACK_CHUNK, pack_body, 0)

    @pl.when(wid == 0)
    def _():
        for i in range(NUM_REL // PACK_CHUNK):
            pack_rows(rel_hbm, relp_hbm, i * PACK_CHUNK, i * PACK_CHUNK)

    flagio_v[pl.ds(0, 16)] = jnp.full((16,), 1, dtype=jnp.int32)
    pltpu.sync_copy(flagio_v, flags_hbm.at[pl.ds(wid * 16, 16)])

    def poll_body(total):
        pltpu.sync_copy(flags_hbm, allf_v)
        acc = [allf_v[pl.ds(w * 16, 16)] for w in range(NW)]
        while len(acc) > 1:
            acc = [acc[2 * i] + acc[2 * i + 1] for i in range(len(acc) // 2)]
        return lax.reduce_sum(acc[0], axes=(0,))

    lax.while_loop(lambda t: t < NW * 16, poll_body, jnp.int32(0))

    pltpu.async_copy(entp_hbm.at[idxneg_v.at[0]], negbuf0, sem0)
    pltpu.async_copy(entp_hbm.at[idxneg_v.at[1]], negbuf1, sem1)
    cg = pltpu.async_copy(entp_hbm.at[idxq0_v], center_v, semg)
    rg = pltpu.async_copy(relp_hbm.at[idxq1_v], rel_v, semg)
    cg.wait()
    rg.wait()

    def center_body(r, carry):
        for c in range(DW // 16):
            sl = pl.ds(c * 16, 16)
            a = plsc.bitcast(center_v[r, sl], jnp.bfloat16)
            b = plsc.bitcast(rel_v[r, sl], jnp.bfloat16)
            center_v[r, sl] = plsc.bitcast(a + b, jnp.int32)
        return carry
    lax.fori_loop(0, BW, center_body, 0)

    posrows_v = rel_v
    pltpu.async_copy(entp_hbm.at[idxpos_v], posrows_v, semg).wait()

    colidx = [jnp.full((16,), k, dtype=jnp.int32) for k in range(16)]

    def l1_rows16(src_v, srow0, cen_rows):
        ss = []
        for j in range(16):
            r = srow0 + j
            if cen_rows is None:
                cvs = [plsc.bitcast(center_v[r, pl.ds(c * 16, 16)], jnp.bfloat16)
                       for c in range(4)]
            else:
                cvs = cen_rows
            ab = [jnp.abs(plsc.bitcast(src_v[r, pl.ds(c * 16, 16)], jnp.bfloat16)
                          - cvs[c]) for c in range(4)]
            up = [plsc.unpack(a, format=plsc.PackFormat.INTERLEAVED) for a in ab]
            d0 = [u[0] + u[1] for u in up]
            d1 = [d0[0] + d0[1], d0[2] + d0[3]]
            ss.append(d1[0] + d1[1])
        for j in range(16):
            trans_v[j, pl.ds(0, 16)] = ss[j]
        cols = [plsc.load_gather(trans_v, [lane, colidx[k]]) for k in range(16)]
        while len(cols) > 1:
            cols = [cols[2 * i] + cols[2 * i + 1] for i in range(len(cols) // 2)]
        return cols[0]

    def pos_body(jg, carry):
        tsum = l1_rows16(posrows_v, jg * 16, None)
        poslog_v[pl.ds(jg * 16, 16)] = GAMMA - tsum
        return carry
    lax.fori_loop(0, BW // 16, pos_body, 0)

    def neg_row(bb, buf):
        cen_rows = [plsc.bitcast(center_v[bb, pl.ds(c * 16, 16)], jnp.bfloat16)
                    for c in range(4)]

        def jg_body(jg, carry):
            tsum = l1_rows16(buf, jg * 16, cen_rows)
            out_v[bb, pl.ds(jg * 16, 16)] = GAMMA - tsum
            return carry
        lax.fori_loop(0, NUM_NEG // 16, jg_body, 0)

    def outer(i, carry):
        bo = i * 2
        for t, (buf, sem) in enumerate(((negbuf0, sem0), (negbuf1, sem1))):
            bb = bo + t
            pltpu.make_async_copy(entp_hbm.at[pl.ds(0, NUM_NEG)], buf, sem).wait()
            neg_row(bb, buf)
            nxt = jnp.minimum(bb + 2, BW - 1)
            pltpu.async_copy(entp_hbm.at[idxneg_v.at[nxt]], buf, sem)
        return carry
    lax.fori_loop(0, BW // 2, outer, 0)

    pltpu.make_async_copy(entp_hbm.at[pl.ds(0, NUM_NEG)], negbuf0, sem0).wait()
    pltpu.make_async_copy(entp_hbm.at[pl.ds(0, NUM_NEG)], negbuf1, sem1).wait()

    pltpu.sync_copy(out_v, nout_hbm.at[pl.ds(base, BW)])
    pltpu.sync_copy(poslog_v, pout_hbm.at[pl.ds(base, BW)])


_mesh = plsc.VectorSubcoreMesh(core_axis_name="c", subcore_axis_name="s",
                               num_cores=NC, num_subcores=NS)

_sc_call = functools.partial(
    pl.kernel,
    out_type=(jax.ShapeDtypeStruct((B,), jnp.float32),
              jax.ShapeDtypeStruct((B, NUM_NEG), jnp.float32),
              jax.ShapeDtypeStruct((NUM_ENT, DW), jnp.int32),
              jax.ShapeDtypeStruct((NUM_REL, DW), jnp.int32),
              jax.ShapeDtypeStruct((NW * 16,), jnp.int32)),
    mesh=_mesh,
    compiler_params=pltpu.CompilerParams(needs_layout_passes=False,
                                         use_tc_tiling_on_sc=False),
    scratch_types=[
        pltpu.VMEM((BW,), jnp.int32),
        pltpu.VMEM((BW,), jnp.int32),
        pltpu.VMEM((BW,), jnp.int32),
        pltpu.VMEM((BW, NUM_NEG), jnp.int32),
        pltpu.VMEM((BW, DW), jnp.int32),
        pltpu.VMEM((BW, DW), jnp.int32),
        pltpu.VMEM((NUM_NEG, DW), jnp.int32),
        pltpu.VMEM((NUM_NEG, DW), jnp.int32),
        pltpu.VMEM((BW, NUM_NEG), jnp.float32),
        pltpu.VMEM((BW,), jnp.float32),
        pltpu.VMEM((16, 17), jnp.float32),
        pltpu.VMEM((PACK_CHUNK, D), jnp.float32),
        pltpu.VMEM((PACK_CHUNK, DW), jnp.int32),
        pltpu.VMEM((16,), jnp.int32),
        pltpu.VMEM((NW * 16,), jnp.int32),
        pltpu.SemaphoreType.DMA,
        pltpu.SemaphoreType.DMA,
        pltpu.SemaphoreType.DMA,
    ],
)(_body)


def kernel(positive_sample, negative_sample, subsampling_weight, queries,
           ent_embedding, rel_embedding):
    del subsampling_weight
    q0 = queries[:, 0].astype(jnp.int32)
    q1 = queries[:, 1].astype(jnp.int32)
    pos = positive_sample.astype(jnp.int32)
    neg = negative_sample.astype(jnp.int32)
    pos_logit, neg_logit, _, _, _ = _sc_call(
        q0, q1, pos, neg, ent_embedding, rel_embedding)
    return jnp.concatenate([pos_logit[:, None], neg_logit], axis=1)


# speedup vs baseline: 1.2174x; 1.2174x over previous
import functools

import jax
import jax.numpy as jnp
from jax import lax
from jax.experimental import pallas as pl
from jax.experimental.pallas import tpu as pltpu
from jax.experimental.pallas import tpu_sc as plsc

B = 4096
NUM_NEG = 128
NUM_ENT = 100000
NUM_REL = 500
D = 128
DW = D // 2
GAMMA = 24.0

NC = 2
NS = 16
NW = NC * NS
BW = B // NW
PACK_CHUNK = 125
ENT_PER_WORKER = NUM_ENT // NW


def _body(q0_hbm, q1_hbm, pos_hbm, neg_hbm, ent_hbm, rel_hbm,
          pout_hbm, nout_hbm, entp_hbm, relp_hbm, flags_hbm,
          idxq0_v, idxq1_v, idxpos_v, idxneg_v,
          center_v, rel_v, negbuf0, negbuf1, out_v, poslog_v, trans_v,
          packin_v, packout_v, flagio_v, allf_v,
          semg, sem0, sem1):
    cc = lax.axis_index("c")
    sid = lax.axis_index("s")
    wid = sid * NC + cc
    base = wid * BW
    lane = lax.iota(jnp.int32, 16)

    flagio_v[pl.ds(0, 16)] = jnp.zeros((16,), jnp.int32)
    pltpu.sync_copy(flagio_v, flags_hbm.at[pl.ds(wid * 16, 16)])

    pltpu.sync_copy(neg_hbm.at[pl.ds(base, BW)], idxneg_v)
    pltpu.sync_copy(q0_hbm.at[pl.ds(base, BW)], idxq0_v)
    pltpu.sync_copy(q1_hbm.at[pl.ds(base, BW)], idxq1_v)
    pltpu.sync_copy(pos_hbm.at[pl.ds(base, BW)], idxpos_v)

    def pack_rows(src_hbm, dst_hbm, src_row0, dst_row0):
        pltpu.sync_copy(src_hbm.at[pl.ds(src_row0, PACK_CHUNK)], packin_v)

        def row_body(i, carry):
            r0 = i * 5
            packed = []
            for j in range(5):
                for c in range(4):
                    a = packin_v[r0 + j, pl.ds(c * 32, 16)]
                    b = packin_v[r0 + j, pl.ds(c * 32 + 16, 16)]
                    p = plsc.pack(a, b, format=plsc.PackFormat.INTERLEAVED)
                    packed.append(plsc.bitcast(p, jnp.int32))
            for j in range(5):
                for c in range(4):
                    packout_v[r0 + j, pl.ds(c * 16, 16)] = packed[4 * j + c]
            return carry
        lax.fori_loop(0, PACK_CHUNK // 5, row_body, 0)
        pltpu.sync_copy(packout_v, dst_hbm.at[pl.ds(dst_row0, PACK_CHUNK)])

    ent_row0 = wid * ENT_PER_WORKER

    def pack_body(i, carry):
        r0 = ent_row0 + i * PACK_CHUNK
        pack_rows(ent_hbm, entp_hbm, r0, r0)
        return carry
    lax.fori_loop(0, ENT_PER_WORKER // PACK_CHUNK, pack_body, 0)

    @pl.when(wid < NUM_REL // PACK_CHUNK)
    def _():
        r0 = wid * PACK_CHUNK
        pack_rows(rel_hbm, relp_hbm, r0, r0)

    flagio_v[pl.ds(0, 16)] = jnp.full((16,), 1, dtype=jnp.int32)
    pltpu.sync_copy(flagio_v, flags_hbm.at[pl.ds(wid * 16, 16)])

    def poll_body(total):
        pltpu.sync_copy(flags_hbm, allf_v)
        acc = [allf_v[pl.ds(w * 16, 16)] for w in range(NW)]
        while len(acc) > 1:
            acc = [acc[2 * i] + acc[2 * i + 1] for i in range(len(acc) // 2)]
        return lax.reduce_sum(acc[0], axes=(0,))

    lax.while_loop(lambda t: t < NW * 16, poll_body, jnp.int32(0))

    pltpu.async_copy(entp_hbm.at[idxneg_v.at[0]], negbuf0, sem0)
    pltpu.async_copy(entp_hbm.at[idxneg_v.at[1]], negbuf1, sem1)
    cg = pltpu.async_copy(entp_hbm.at[idxq0_v], center_v, semg)
    rg = pltpu.async_copy(relp_hbm.at[idxq1_v], rel_v, semg)
    cg.wait()
    rg.wait()

    def center_body(r, carry):
        for c in range(DW // 16):
            sl = pl.ds(c * 16, 16)
            a = plsc.bitcast(center_v[r, sl], jnp.bfloat16)
            b = plsc.bitcast(rel_v[r, sl], jnp.bfloat16)
            center_v[r, sl] = plsc.bitcast(a + b, jnp.int32)
        return carry
    lax.fori_loop(0, BW, center_body, 0)

    posrows_v = rel_v
    pltpu.async_copy(entp_hbm.at[idxpos_v], posrows_v, semg).wait()

    colidx = [jnp.full((16,), k, dtype=jnp.int32) for k in range(16)]

    def l1_rows16(src_v, srow0, cen_rows):
        ss = []
        for j in range(16):
            r = srow0 + j
            if cen_rows is None:
                cvs = [plsc.bitcast(center_v[r, pl.ds(c * 16, 16)], jnp.bfloat16)
                       for c in range(4)]
            else:
                cvs = cen_rows
            ab = [jnp.abs(plsc.bitcast(src_v[r, pl.ds(c * 16, 16)], jnp.bfloat16)
                          - cvs[c]) for c in range(4)]
            up = [plsc.unpack(a, format=plsc.PackFormat.INTERLEAVED) for a in ab]
            d0 = [u[0] + u[1] for u in up]
            d1 = [d0[0] + d0[1], d0[2] + d0[3]]
            ss.append(d1[0] + d1[1])
        for j in range(16):
            trans_v[j, pl.ds(0, 16)] = ss[j]
        cols = [plsc.load_gather(trans_v, [lane, colidx[k]]) for k in range(16)]
        while len(cols) > 1:
            cols = [cols[2 * i] + cols[2 * i + 1] for i in range(len(cols) // 2)]
        return cols[0]

    def pos_body(jg, carry):
        tsum = l1_rows16(posrows_v, jg * 16, None)
        poslog_v[pl.ds(jg * 16, 16)] = GAMMA - tsum
        return carry
    lax.fori_loop(0, BW // 16, pos_body, 0)

    def neg_row(bb, buf):
        cen_rows = [plsc.bitcast(center_v[bb, pl.ds(c * 16, 16)], jnp.bfloat16)
                    for c in range(4)]

        def jg_body(jg, carry):
            tsum = l1_rows16(buf, jg * 16, cen_rows)
            out_v[bb, pl.ds(jg * 16, 16)] = GAMMA - tsum
            return carry
        lax.fori_loop(0, NUM_NEG // 16, jg_body, 0)

    def outer(i, carry):
        bo = i * 2
        for t, (buf, sem) in enumerate(((negbuf0, sem0), (negbuf1, sem1))):
            bb = bo + t
            pltpu.make_async_copy(entp_hbm.at[pl.ds(0, NUM_NEG)], buf, sem).wait()
            neg_row(bb, buf)
            nxt = jnp.minimum(bb + 2, BW - 1)
            pltpu.async_copy(entp_hbm.at[idxneg_v.at[nxt]], buf, sem)
        return carry
    lax.fori_loop(0, BW // 2, outer, 0)

    pltpu.make_async_copy(entp_hbm.at[pl.ds(0, NUM_NEG)], negbuf0, sem0).wait()
    pltpu.make_async_copy(entp_hbm.at[pl.ds(0, NUM_NEG)], negbuf1, sem1).wait()

    pltpu.sync_copy(out_v, nout_hbm.at[pl.ds(base, BW)])
    pltpu.sync_copy(poslog_v, pout_hbm.at[pl.ds(base, BW)])


_mesh = plsc.VectorSubcoreMesh(core_axis_name="c", subcore_axis_name="s",
                               num_cores=NC, num_subcores=NS)

_sc_call = functools.partial(
    pl.kernel,
    out_type=(jax.ShapeDtypeStruct((B,), jnp.float32),
              jax.ShapeDtypeStruct((B, NUM_NEG), jnp.float32),
              jax.ShapeDtypeStruct((NUM_ENT, DW), jnp.int32),
              jax.ShapeDtypeStruct((NUM_REL, DW), jnp.int32),
              jax.ShapeDtypeStruct((NW * 16,), jnp.int32)),
    mesh=_mesh,
    compiler_params=pltpu.CompilerParams(needs_layout_passes=False,
                                         use_tc_tiling_on_sc=False),
    scratch_types=[
        pltpu.VMEM((BW,), jnp.int32),
        pltpu.VMEM((BW,), jnp.int32),
        pltpu.VMEM((BW,), jnp.int32),
        pltpu.VMEM((BW, NUM_NEG), jnp.int32),
        pltpu.VMEM((BW, DW), jnp.int32),
        pltpu.VMEM((BW, DW), jnp.int32),
        pltpu.VMEM((NUM_NEG, DW), jnp.int32),
        pltpu.VMEM((NUM_NEG, DW), jnp.int32),
        pltpu.VMEM((BW, NUM_NEG), jnp.float32),
        pltpu.VMEM((BW,), jnp.float32),
        pltpu.VMEM((16, 17), jnp.float32),
        pltpu.VMEM((PACK_CHUNK, D), jnp.float32),
        pltpu.VMEM((PACK_CHUNK, DW), jnp.int32),
        pltpu.VMEM((16,), jnp.int32),
        pltpu.VMEM((NW * 16,), jnp.int32),
        pltpu.SemaphoreType.DMA,
        pltpu.SemaphoreType.DMA,
        pltpu.SemaphoreType.DMA,
    ],
)(_body)


def kernel(positive_sample, negative_sample, subsampling_weight, queries,
           ent_embedding, rel_embedding):
    del subsampling_weight
    q0 = queries[:, 0].astype(jnp.int32)
    q1 = queries[:, 1].astype(jnp.int32)
    pos = positive_sample.astype(jnp.int32)
    neg = negative_sample.astype(jnp.int32)
    pos_logit, neg_logit, _, _, _ = _sc_call(
        q0, q1, pos, neg, ent_embedding, rel_embedding)
    return jnp.concatenate([pos_logit[:, None], neg_logit], axis=1)


# speedup vs baseline: 1.4012x; 1.1510x over previous
import functools

import jax
import jax.numpy as jnp
from jax import lax
from jax.experimental import pallas as pl
from jax.experimental.pallas import tpu as pltpu
from jax.experimental.pallas import tpu_sc as plsc

B = 4096
NUM_NEG = 128
NUM_ENT = 100000
NUM_REL = 500
D = 128
DW = D // 2
GAMMA = 24.0

NC = 2
NS = 16
NW = NC * NS
BW = B // NW
PACK_CHUNK = 125
ENT_PER_WORKER = NUM_ENT // NW


def _body(q0_hbm, q1_hbm, pos_hbm, neg_hbm, ent_hbm, rel_hbm,
          pout_hbm, nout_hbm, entp_hbm, relp_hbm, flags_hbm,
          idxq0_v, idxq1_v, idxpos_v, idxneg_v,
          center_v, rel_v, negbuf0, negbuf1, out_v, poslog_v, trans_v,
          packin_v, packin2_v, packout_v, flagio_v, allf_v,
          semg, sem0, sem1, semp0, semp1):
    cc = lax.axis_index("c")
    sid = lax.axis_index("s")
    wid = sid * NC + cc
    base = wid * BW
    lane = lax.iota(jnp.int32, 16)

    flagio_v[pl.ds(0, 16)] = jnp.zeros((16,), jnp.int32)
    pltpu.sync_copy(flagio_v, flags_hbm.at[pl.ds(wid * 16, 16)])

    pltpu.sync_copy(neg_hbm.at[pl.ds(base, BW)], idxneg_v)
    pltpu.sync_copy(q0_hbm.at[pl.ds(base, BW)], idxq0_v)
    pltpu.sync_copy(q1_hbm.at[pl.ds(base, BW)], idxq1_v)
    pltpu.sync_copy(pos_hbm.at[pl.ds(base, BW)], idxpos_v)

    def pack_compute(pin, dst_hbm, dst_row0):
        def row_body(i, carry):
            r0 = i * 5
            packed = []
            for j in range(5):
                for c in range(4):
                    a = pin[r0 + j, pl.ds(c * 32, 16)]
                    b = pin[r0 + j, pl.ds(c * 32 + 16, 16)]
                    p = plsc.pack(a, b, format=plsc.PackFormat.INTERLEAVED)
                    packed.append(plsc.bitcast(p, jnp.int32))
            for j in range(5):
                for c in range(4):
                    packout_v[r0 + j, pl.ds(c * 16, 16)] = packed[4 * j + c]
            return carry
        lax.fori_loop(0, PACK_CHUNK // 5, row_body, 0)
        pltpu.sync_copy(packout_v, dst_hbm.at[pl.ds(dst_row0, PACK_CHUNK)])


    ent_row0 = wid * ENT_PER_WORKER
    nch = ENT_PER_WORKER // PACK_CHUNK
    last = nch - 1

    def chunk_src(k):
        return ent_hbm.at[pl.ds(ent_row0 + k * PACK_CHUNK, PACK_CHUNK)]

    pltpu.async_copy(chunk_src(0), packin_v, semp0)
    pltpu.async_copy(chunk_src(1), packin2_v, semp1)

    def pack_body(i, carry):
        k0 = i * 2
        for t, (pin, semp) in enumerate(((packin_v, semp0), (packin2_v, semp1))):
            k = k0 + t
            pltpu.make_async_copy(chunk_src(0), pin, semp).wait()
            pack_compute(pin, entp_hbm, ent_row0 + k * PACK_CHUNK)
            nxt = jnp.minimum(k + 2, last)
            pltpu.async_copy(chunk_src(nxt), pin, semp)
        return carry
    lax.fori_loop(0, (nch - 1) // 2, pack_body, 0)
    pltpu.make_async_copy(chunk_src(0), packin_v, semp0).wait()
    pack_compute(packin_v, entp_hbm, ent_row0 + last * PACK_CHUNK)
    pltpu.make_async_copy(chunk_src(0), packin2_v, semp1).wait()

    @pl.when(wid < NUM_REL // PACK_CHUNK)
    def _():
        r0 = wid * PACK_CHUNK
        pltpu.sync_copy(rel_hbm.at[pl.ds(r0, PACK_CHUNK)], packin_v)
        pack_compute(packin_v, relp_hbm, r0)

    flagio_v[pl.ds(0, 16)] = jnp.full((16,), 1, dtype=jnp.int32)
    pltpu.sync_copy(flagio_v, flags_hbm.at[pl.ds(wid * 16, 16)])

    def poll_body(total):
        pltpu.sync_copy(flags_hbm, allf_v)
        acc = [allf_v[pl.ds(w * 16, 16)] for w in range(NW)]
        while len(acc) > 1:
            acc = [acc[2 * i] + acc[2 * i + 1] for i in range(len(acc) // 2)]
        return lax.reduce_sum(acc[0], axes=(0,))

    lax.while_loop(lambda t: t < NW * 16, poll_body, jnp.int32(0))

    pltpu.async_copy(entp_hbm.at[idxneg_v.at[0]], negbuf0, sem0)
    pltpu.async_copy(entp_hbm.at[idxneg_v.at[1]], negbuf1, sem1)
    cg = pltpu.async_copy(entp_hbm.at[idxq0_v], center_v, semg)
    rg = pltpu.async_copy(relp_hbm.at[idxq1_v], rel_v, semg)
    cg.wait()
    rg.wait()

    def center_body(r, carry):
        for c in range(DW // 16):
            sl = pl.ds(c * 16, 16)
            a = plsc.bitcast(center_v[r, sl], jnp.bfloat16)
            b = plsc.bitcast(rel_v[r, sl], jnp.bfloat16)
            center_v[r, sl] = plsc.bitcast(a + b, jnp.int32)
        return carry
    lax.fori_loop(0, BW, center_body, 0)

    posrows_v = rel_v
    pltpu.async_copy(entp_hbm.at[idxpos_v], posrows_v, semg).wait()

    colidx = [jnp.full((16,), k, dtype=jnp.int32) for k in range(16)]

    def l1_rows16(src_v, srow0, cen_rows):
        ss = []
        for j in range(16):
            r = srow0 + j
            if cen_rows is None:
                cvs = [plsc.bitcast(center_v[r, pl.ds(c * 16, 16)], jnp.bfloat16)
                       for c in range(4)]
            else:
                cvs = cen_rows
            ab = [jnp.abs(plsc.bitcast(src_v[r, pl.ds(c * 16, 16)], jnp.bfloat16)
                          - cvs[c]) for c in range(4)]
            up = [plsc.unpack(a, format=plsc.PackFormat.INTERLEAVED) for a in ab]
            d0 = [u[0] + u[1] for u in up]
            d1 = [d0[0] + d0[1], d0[2] + d0[3]]
            ss.append(d1[0] + d1[1])
        for j in range(16):
            trans_v[j, pl.ds(0, 16)] = ss[j]
        cols = [plsc.load_gather(trans_v, [lane, colidx[k]]) for k in range(16)]
        while len(cols) > 1:
            cols = [cols[2 * i] + cols[2 * i + 1] for i in range(len(cols) // 2)]
        return cols[0]

    def pos_body(jg, carry):
        tsum = l1_rows16(posrows_v, jg * 16, None)
        poslog_v[pl.ds(jg * 16, 16)] = GAMMA - tsum
        return carry
    lax.fori_loop(0, BW // 16, pos_body, 0)

    def neg_row(bb, buf):
        cen_rows = [plsc.bitcast(center_v[bb, pl.ds(c * 16, 16)], jnp.bfloat16)
                    for c in range(4)]

        def jg_body(jg, carry):
            tsum = l1_rows16(buf, jg * 16, cen_rows)
            out_v[bb, pl.ds(jg * 16, 16)] = GAMMA - tsum
            return carry
        lax.fori_loop(0, NUM_NEG // 16, jg_body, 0)

    def outer(i, carry):
        bo = i * 2
        for t, (buf, sem) in enumerate(((negbuf0, sem0), (negbuf1, sem1))):
            bb = bo + t
            pltpu.make_async_copy(entp_hbm.at[pl.ds(0, NUM_NEG)], buf, sem).wait()
            neg_row(bb, buf)
            nxt = jnp.minimum(bb + 2, BW - 1)
            pltpu.async_copy(entp_hbm.at[idxneg_v.at[nxt]], buf, sem)
        return carry
    lax.fori_loop(0, BW // 2, outer, 0)

    pltpu.make_async_copy(entp_hbm.at[pl.ds(0, NUM_NEG)], negbuf0, sem0).wait()
    pltpu.make_async_copy(entp_hbm.at[pl.ds(0, NUM_NEG)], negbuf1, sem1).wait()

    pltpu.sync_copy(out_v, nout_hbm.at[pl.ds(base, BW)])
    pltpu.sync_copy(poslog_v, pout_hbm.at[pl.ds(base, BW)])


_mesh = plsc.VectorSubcoreMesh(core_axis_name="c", subcore_axis_name="s",
                               num_cores=NC, num_subcores=NS)

_sc_call = functools.partial(
    pl.kernel,
    out_type=(jax.ShapeDtypeStruct((B,), jnp.float32),
              jax.ShapeDtypeStruct((B, NUM_NEG), jnp.float32),
              jax.ShapeDtypeStruct((NUM_ENT, DW), jnp.int32),
              jax.ShapeDtypeStruct((NUM_REL, DW), jnp.int32),
              jax.ShapeDtypeStruct((NW * 16,), jnp.int32)),
    mesh=_mesh,
    compiler_params=pltpu.CompilerParams(needs_layout_passes=False,
                                         use_tc_tiling_on_sc=False),
    scratch_types=[
        pltpu.VMEM((BW,), jnp.int32),
        pltpu.VMEM((BW,), jnp.int32),
        pltpu.VMEM((BW,), jnp.int32),
        pltpu.VMEM((BW, NUM_NEG), jnp.int32),
        pltpu.VMEM((BW, DW), jnp.int32),
        pltpu.VMEM((BW, DW), jnp.int32),
        pltpu.VMEM((NUM_NEG, DW), jnp.int32),
        pltpu.VMEM((NUM_NEG, DW), jnp.int32),
        pltpu.VMEM((BW, NUM_NEG), jnp.float32),
        pltpu.VMEM((BW,), jnp.float32),
        pltpu.VMEM((16, 17), jnp.float32),
        pltpu.VMEM((PACK_CHUNK, D), jnp.float32),
        pltpu.VMEM((PACK_CHUNK, D), jnp.float32),
        pltpu.VMEM((PACK_CHUNK, DW), jnp.int32),
        pltpu.VMEM((16,), jnp.int32),
        pltpu.VMEM((NW * 16,), jnp.int32),
        pltpu.SemaphoreType.DMA,
        pltpu.SemaphoreType.DMA,
        pltpu.SemaphoreType.DMA,
        pltpu.SemaphoreType.DMA,
        pltpu.SemaphoreType.DMA,
    ],
)(_body)


def kernel(positive_sample, negative_sample, subsampling_weight, queries,
           ent_embedding, rel_embedding):
    del subsampling_weight
    q0 = queries[:, 0].astype(jnp.int32)
    q1 = queries[:, 1].astype(jnp.int32)
    pos = positive_sample.astype(jnp.int32)
    neg = negative_sample.astype(jnp.int32)
    pos_logit, neg_logit, _, _, _ = _sc_call(
        q0, q1, pos, neg, ent_embedding, rel_embedding)
    return jnp.concatenate([pos_logit[:, None], neg_logit], axis=1)
